# baseline (speedup 1.0000x reference)
import functools

import numpy as np

import jax
import jax.numpy as jnp
from jax import lax
from jax.experimental import pallas as pl
from jax.experimental.pallas import tpu as pltpu
from jax.experimental.pallas import tpu_sc as plsc

N_NODES = 10000
NP = 10240
TRASH = N_NODES
E_RAW = 320000
E_SL = E_RAW + N_NODES
EP = 344064
GBA = 64
RB = EP // GBA
NC, NS = 2, 16
ET = EP // (NC * NS)
RA2 = RB // (NC * NS)
NPT = NP // NS
NB = 1024
HEADS = 8
CE = 768


def _tc1_body(x_ref, w_ref, s_ref, d_ref, h_ref, as_ref, ad_ref):
    h = jnp.dot(x_ref[...], w_ref[...], preferred_element_type=jnp.float32)
    h_ref[...] = h
    as_ref[...] = jnp.dot(h, s_ref[...], preferred_element_type=jnp.float32)
    ad_ref[...] = jnp.dot(h, d_ref[...], preferred_element_type=jnp.float32)


def _tc2_body(p0_ref, p1_ref, b_ref, w_ref, s_ref, d_ref, h_ref, as_ref, ad_ref):
    xin = p0_ref[...] + p1_ref[...] + b_ref[...]
    xin = jnp.where(xin > 0, xin, jnp.exp(jnp.minimum(xin, 0.0)) - 1.0)
    h = jnp.dot(xin, w_ref[...], preferred_element_type=jnp.float32)
    h_ref[...] = h.astype(jnp.bfloat16)
    as_ref[...] = jnp.dot(h, s_ref[...], preferred_element_type=jnp.float32)
    ad_ref[...] = jnp.dot(h, d_ref[...], preferred_element_type=jnp.float32)


def _rep(shape):
    return pl.BlockSpec(shape, lambda i: (0, 0))


def _row(shape):
    return pl.BlockSpec(shape, lambda i: (i, 0))


_tc1 = pl.pallas_call(
    _tc1_body,
    grid=(NP // NB,),
    in_specs=[_row((NB, 128)), _rep((128, 128)), _rep((128, 16)), _rep((128, 16))],
    out_specs=[_row((NB, 128)), _row((NB, 16)), _row((NB, 16))],
    out_shape=[
        jax.ShapeDtypeStruct((NP, 128), jnp.float32),
        jax.ShapeDtypeStruct((NP, 16), jnp.float32),
        jax.ShapeDtypeStruct((NP, 16), jnp.float32),
    ],
)

_tc2 = pl.pallas_call(
    _tc2_body,
    grid=(NP // NB,),
    in_specs=[_row((NB, 128)), _row((NB, 128)), _rep((1, 128)), _rep((128, 1024)),
              _rep((1024, 16)), _rep((1024, 16))],
    out_specs=[_row((NB, 1024)), _row((NB, 16)), _row((NB, 16))],
    out_shape=[
        jax.ShapeDtypeStruct((NP, 1024), jnp.bfloat16),
        jax.ShapeDtypeStruct((NP, 16), jnp.float32),
        jax.ShapeDtypeStruct((NP, 16), jnp.float32),
    ],
)


def _rden_body_factory(inv_h):
    def body(d0_ref, d1_ref, o_ref):
        o_ref[...] = inv_h / (d0_ref[...] + d1_ref[...] + 1e-16)
    return body


@functools.lru_cache(maxsize=None)
def _tc_rden(inv_h):
    return pl.pallas_call(
        _rden_body_factory(inv_h),
        grid=(1,),
        in_specs=[_rep((NP, 16)), _rep((NP, 16))],
        out_specs=_rep((NP, 16)),
        out_shape=jax.ShapeDtypeStruct((NP, 16), jnp.float32),
    )


def _sc_mesh():
    return plsc.VectorSubcoreMesh(core_axis_name="c", subcore_axis_name="s",
                                  num_cores=NC, num_subcores=NS)


@functools.lru_cache(maxsize=None)
def _make_sc_att():
    @functools.partial(
        pl.kernel,
        mesh=_sc_mesh(),
        compiler_params=pltpu.CompilerParams(use_tc_tiling_on_sc=False),
        out_type=(
            jax.ShapeDtypeStruct((EP, 16), jnp.float32),
            jax.ShapeDtypeStruct((NC, NP, 16), jnp.float32),
        ),
        scratch_types=[
            pltpu.VMEM((RA2, GBA), jnp.int32),
            pltpu.VMEM((RA2, GBA), jnp.int32),
            pltpu.VMEM((GBA, 16), jnp.float32),
            pltpu.VMEM((GBA, 16), jnp.float32),
            pltpu.VMEM((GBA, 16), jnp.float32),
            pltpu.VMEM((GBA, 16), jnp.float32),
            pltpu.VMEM((GBA, 16), jnp.float32),
            pltpu.VMEM((GBA, 16), jnp.float32),
            pltpu.VMEM_SHARED((NP, 16), jnp.float32),
            pltpu.SemaphoreType.DMA,
            pltpu.SemaphoreType.DMA,
        ],
    )
    def sc_att(src_hbm, dst_hbm, asrc_hbm, adst_hbm, w_hbm, den_hbm,
               src_v, dst_v, asg0, adg0, asg1, adg1, coef0, coef1,
               den_sh, sem0, sem1):
        c = lax.axis_index("c")
        s = lax.axis_index("s")
        base_n = s * NPT
        tile = c * NS + s
        row0 = tile * RA2
        slots = ((asg0, adg0, coef0, sem0), (asg1, adg1, coef1, sem1))

        def zero_body(i, _):
            asg0[i] = jnp.zeros((16,), jnp.float32)
            return 0
        lax.fori_loop(0, GBA, zero_body, 0)
        for m in range(NPT // GBA):
            pltpu.sync_copy(asg0, den_sh.at[pl.ds(base_n + m * GBA, GBA)])
        plsc.subcore_barrier()

        pltpu.sync_copy(src_hbm.at[pl.ds(row0, RA2)], src_v)
        pltpu.sync_copy(dst_hbm.at[pl.ds(row0, RA2)], dst_v)

        def issue(j, b):
            asg, adg, _, sem = slots[b]
            pltpu.async_copy(asrc_hbm.at[src_v.at[j]], asg, sem)
            pltpu.async_copy(adst_hbm.at[dst_v.at[j]], adg, sem)

        def process(j, b):
            asg, adg, coef, sem = slots[b]
            pltpu.make_async_copy(asrc_hbm.at[src_v.at[j]], asg, sem).wait()
            pltpu.make_async_copy(adst_hbm.at[dst_v.at[j]], adg, sem).wait()

            def att(i, _):
                e = asg[i] + adg[i]
                e = jnp.where(e >= 0.0, e, 0.2 * e)
                coef[i] = jnp.exp(e)
                return 0
            lax.fori_loop(0, GBA, att, 0)
            pltpu.sync_copy(coef, w_hbm.at[pl.ds((row0 + j) * GBA, GBA)])
            pltpu.sync_copy(coef, den_sh.at[dst_v.at[j]], add=True)

            @pl.when(j + 2 < RA2)
            def _():
                issue(j + 2, b)

        issue(0, 0)
        issue(1, 1)

        def pair(jp, _):
            process(2 * jp, 0)
            process(2 * jp + 1, 1)
            return 0
        lax.fori_loop(0, RA2 // 2, pair, 0)
        plsc.subcore_barrier()

        pltpu.sync_copy(den_sh.at[pl.ds(base_n, NPT)],
                        den_hbm.at[c].at[pl.ds(base_n, NPT)])

    return sc_att


def _make_sc_agg(hc, gb, ch, bf16_feat=False):
    feat_w = HEADS * hc
    fdt = jnp.bfloat16 if bf16_feat else jnp.float32
    rows_t = ET // gb
    n_ch = rows_t // ch

    @functools.partial(
        pl.kernel,
        mesh=_sc_mesh(),
        compiler_params=pltpu.CompilerParams(use_tc_tiling_on_sc=False,
                                             needs_layout_passes=False),
        out_type=jax.ShapeDtypeStruct((NC, NP, 128), jnp.float32),
        scratch_types=[
            pltpu.VMEM((ch, gb), jnp.int32),
            pltpu.VMEM((ch, gb), jnp.int32),
            pltpu.VMEM((gb, 16), jnp.float32),
            pltpu.VMEM((gb, 16), jnp.float32),
            pltpu.VMEM((gb, 16), jnp.float32),
            pltpu.VMEM((gb, 16), jnp.float32),
            pltpu.VMEM((gb, feat_w), fdt),
            pltpu.VMEM((gb, feat_w), fdt),
            pltpu.VMEM((gb, 128), jnp.float32),
            pltpu.VMEM((gb, 128), jnp.float32),
            pltpu.VMEM_SHARED((NP, 128), jnp.float32),
            pltpu.SemaphoreType.DMA,
            pltpu.SemaphoreType.DMA,
            pltpu.SemaphoreType.DMA,
            pltpu.SemaphoreType.DMA,
        ],
    )
    def sc_agg(src_hbm, dst_hbm, w_hbm, rden_hbm, feat_hbm, out_hbm,
               src_v, dst_v, wv0, wv1, rdg0, rdg1, g0, g1, ob0, ob1,
               out_sh, sem0, sem1, ssem0, ssem1):
        c = lax.axis_index("c")
        s = lax.axis_index("s")
        base_n = s * NPT
        tile = c * NS + s
        e0 = tile * ET
        slots = ((wv0, rdg0, g0, ob0, sem0, ssem0),
                 (wv1, rdg1, g1, ob1, sem1, ssem1))

        def zero_body(i, _):
            z = jnp.zeros((16,), jnp.float32)
            for k in range(8):
                ob0[i, pl.ds(16 * k, 16)] = z
            return 0
        lax.fori_loop(0, gb, zero_body, 0)
        for m in range(NPT // gb):
            pltpu.sync_copy(ob0, out_sh.at[pl.ds(base_n + m * gb, gb)])
        plsc.subcore_barrier()

        def issue(jj, cc, b):
            wv, rdg, g, _, sem, _ = slots[b]
            ebase = e0 + (cc * ch + jj) * gb
            pltpu.async_copy(w_hbm.at[pl.ds(ebase, gb)], wv, sem)
            pltpu.async_copy(rden_hbm.at[dst_v.at[jj]], rdg, sem)
            pltpu.async_copy(feat_hbm.at[src_v.at[jj]], g, sem)

        def process(jj, cc, b):
            wv, rdg, g, ob, sem, ssem = slots[b]
            dr = dst_v.at[jj]
            ebase = e0 + (cc * ch + jj) * gb
            pltpu.make_async_copy(w_hbm.at[pl.ds(ebase, gb)], wv, sem).wait()
            pltpu.make_async_copy(rden_hbm.at[dr], rdg, sem).wait()
            pltpu.make_async_copy(feat_hbm.at[src_v.at[jj]], g, sem).wait()

            @pl.when(jj >= 2)
            def _():
                pltpu.make_async_copy(ob, out_sh.at[dr], ssem).wait()

            def edge(i, _):
                cv = wv[i] * rdg[i]
                cs = [cv[h] for h in range(HEADS)]
                if hc == 16:
                    for k in range(8):
                        ob[i, pl.ds(16 * k, 16)] = g[i, pl.ds(16 * k, 16)] * cs[k]
                elif not bf16_feat:
                    for k in range(8):
                        acc = g[i, pl.ds(16 * k, 16)] * cs[0]
                        for h in range(1, HEADS):
                            acc = acc + g[i, pl.ds(128 * h + 16 * k, 16)] * cs[h]
                        ob[i, pl.ds(16 * k, 16)] = acc
                else:
                    for m in range(4):
                        lo, hi = plsc.unpack(
                            g[i, pl.ds(32 * m, 32)],
                            format=plsc.PackFormat.INTERLEAVED)
                        acc_a = lo * cs[0]
                        acc_b = hi * cs[0]
                        for h in range(1, HEADS):
                            lo, hi = plsc.unpack(
                                g[i, pl.ds(128 * h + 32 * m, 32)],
                                format=plsc.PackFormat.INTERLEAVED)
                            acc_a = acc_a + lo * cs[h]
                            acc_b = acc_b + hi * cs[h]
                        ob[i, pl.ds(32 * m, 16)] = acc_a
                        ob[i, pl.ds(32 * m + 16, 16)] = acc_b
                return 0
            lax.fori_loop(0, gb, edge, 0)
            pltpu.async_copy(ob, out_sh.at[dr], ssem, add=True)

            @pl.when(jj + 2 < ch)
            def _():
                issue(jj + 2, cc, b)

        def chunk(cc, _):
            pltpu.sync_copy(src_hbm.at[pl.ds(tile * rows_t + cc * ch, ch)], src_v)
            pltpu.sync_copy(dst_hbm.at[pl.ds(tile * rows_t + cc * ch, ch)], dst_v)
            issue(0, cc, 0)
            issue(1, cc, 1)

            def pair(jp, _):
                process(2 * jp, cc, 0)
                process(2 * jp + 1, cc, 1)
                return 0
            lax.fori_loop(0, ch // 2, pair, 0)
            pltpu.make_async_copy(ob0, out_sh.at[dst_v.at[ch - 2]], ssem0).wait()
            pltpu.make_async_copy(ob1, out_sh.at[dst_v.at[ch - 1]], ssem1).wait()
            return 0
        lax.fori_loop(0, n_ch, chunk, 0)
        plsc.subcore_barrier()

        pltpu.sync_copy(out_sh.at[pl.ds(base_n, NPT)],
                        out_hbm.at[c].at[pl.ds(base_n, NPT)])

    return sc_agg


@functools.lru_cache(maxsize=None)
def _sc_kernels(layer):
    if layer == 1:
        return _make_sc_att(), _make_sc_agg(16, 32, 48)
    return _make_sc_att(), _make_sc_agg(128, 32, 48, bf16_feat=True)


_D2 = HEADS * 128
_r = np.arange(_D2)
_PERM2 = (_r // 32) * 32 + np.where(_r % 2 == 0, (_r % 32) // 2,
                                    16 + (_r % 32) // 2)


def _block_diag(att, hc):
    d = HEADS * hc
    rows = jnp.arange(d)
    return jnp.zeros((d, 16), jnp.float32).at[rows, rows // hc].set(att.reshape(d))


@jax.jit
def kernel(x, edge_index, W1, att_src1, att_dst1, b1, W2, att_src2, att_dst2, b2):
    xp = jnp.zeros((NP, 128), jnp.float32).at[:N_NODES].set(x)
    loop = jnp.arange(N_NODES, dtype=jnp.int32)
    pad = TRASH + jnp.arange(EP - E_SL, dtype=jnp.int32) % (NP - N_NODES)
    src = jnp.concatenate([edge_index[0], loop, pad]).reshape(RB, GBA)
    dst = jnp.concatenate([edge_index[1], loop, pad]).reshape(RB, GBA)
    src32 = src.reshape(RB * 2, 32)
    dst32 = dst.reshape(RB * 2, 32)

    att1, agg1 = _sc_kernels(1)
    att2, agg2 = _sc_kernels(2)

    h1, a1s, a1d = _tc1(xp, W1, _block_diag(att_src1, 16), _block_diag(att_dst1, 16))
    w1, d1 = att1(src, dst, a1s, a1d)
    rd1 = _tc_rden(1.0)(d1[0], d1[1])
    p1 = agg1(src32, dst32, w1, rd1, h1)
    h2, a2s, a2d = _tc2(p1[0], p1[1], b1.reshape(1, 128), W2[:, _PERM2],
                        _block_diag(att_src2.reshape(_D2)[_PERM2], 128),
                        _block_diag(att_dst2.reshape(_D2)[_PERM2], 128))
    w2, d2 = att2(src, dst, a2s, a2d)
    rd2 = _tc_rden(1.0 / HEADS)(d2[0], d2[1])
    p2 = agg2(src32, dst32, w2, rd2, h2)
    return p2[0, :N_NODES] + p2[1, :N_NODES] + b2


# speedup vs baseline: 1.1305x; 1.1305x over previous
import functools

import numpy as np

import jax
import jax.numpy as jnp
from jax import lax
from jax.experimental import pallas as pl
from jax.experimental.pallas import tpu as pltpu
from jax.experimental.pallas import tpu_sc as plsc

N_NODES = 10000
NP = 10240
TRASH = N_NODES
E_RAW = 320000
E_SL = E_RAW + N_NODES
EP = 344064
GBA = 64
RB = EP // GBA
NC, NS = 2, 16
ET = EP // (NC * NS)
RA2 = RB // (NC * NS)
NPT = NP // NS
NB = 1024
HEADS = 8
CE = 768


def _tc1_body(x_ref, w_ref, s_ref, d_ref, h_ref, as_ref, ad_ref):
    h = jnp.dot(x_ref[...], w_ref[...], preferred_element_type=jnp.float32)
    h_ref[...] = h
    as_ref[...] = jnp.dot(h, s_ref[...], preferred_element_type=jnp.float32)
    ad_ref[...] = jnp.dot(h, d_ref[...], preferred_element_type=jnp.float32)


def _tc2_body(p0_ref, p1_ref, b_ref, w_ref, s_ref, d_ref, h_ref, as_ref, ad_ref):
    xin = p0_ref[...] + p1_ref[...] + b_ref[...]
    xin = jnp.where(xin > 0, xin, jnp.exp(jnp.minimum(xin, 0.0)) - 1.0)
    h = jnp.dot(xin, w_ref[...], preferred_element_type=jnp.float32)
    h_ref[...] = h.astype(jnp.bfloat16)
    as_ref[...] = jnp.dot(h, s_ref[...], preferred_element_type=jnp.float32)
    ad_ref[...] = jnp.dot(h, d_ref[...], preferred_element_type=jnp.float32)


def _rep(shape):
    return pl.BlockSpec(shape, lambda i: (0, 0))


def _row(shape):
    return pl.BlockSpec(shape, lambda i: (i, 0))


_tc1 = pl.pallas_call(
    _tc1_body,
    grid=(NP // NB,),
    in_specs=[_row((NB, 128)), _rep((128, 128)), _rep((128, 16)), _rep((128, 16))],
    out_specs=[_row((NB, 128)), _row((NB, 16)), _row((NB, 16))],
    out_shape=[
        jax.ShapeDtypeStruct((NP, 128), jnp.float32),
        jax.ShapeDtypeStruct((NP, 16), jnp.float32),
        jax.ShapeDtypeStruct((NP, 16), jnp.float32),
    ],
)

_tc2 = pl.pallas_call(
    _tc2_body,
    grid=(NP // NB,),
    in_specs=[_row((NB, 128)), _row((NB, 128)), _rep((1, 128)), _rep((128, 1024)),
              _rep((1024, 16)), _rep((1024, 16))],
    out_specs=[_row((NB, 1024)), _row((NB, 16)), _row((NB, 16))],
    out_shape=[
        jax.ShapeDtypeStruct((NP, 1024), jnp.bfloat16),
        jax.ShapeDtypeStruct((NP, 16), jnp.float32),
        jax.ShapeDtypeStruct((NP, 16), jnp.float32),
    ],
)


def _rden_body_factory(inv_h):
    def body(d0_ref, d1_ref, o_ref):
        o_ref[...] = inv_h / (d0_ref[...] + d1_ref[...] + 1e-16)
    return body


@functools.lru_cache(maxsize=None)
def _tc_rden(inv_h):
    return pl.pallas_call(
        _rden_body_factory(inv_h),
        grid=(1,),
        in_specs=[_rep((NP, 16)), _rep((NP, 16))],
        out_specs=_rep((NP, 16)),
        out_shape=jax.ShapeDtypeStruct((NP, 16), jnp.float32),
    )


def _sc_mesh():
    return plsc.VectorSubcoreMesh(core_axis_name="c", subcore_axis_name="s",
                                  num_cores=NC, num_subcores=NS)


@functools.lru_cache(maxsize=None)
def _make_sc_att():
    @functools.partial(
        pl.kernel,
        mesh=_sc_mesh(),
        compiler_params=pltpu.CompilerParams(use_tc_tiling_on_sc=False),
        out_type=(
            jax.ShapeDtypeStruct((EP, 16), jnp.float32),
            jax.ShapeDtypeStruct((NC, NP, 16), jnp.float32),
        ),
        scratch_types=[
            pltpu.VMEM((RA2, GBA), jnp.int32),
            pltpu.VMEM((RA2, GBA), jnp.int32),
            pltpu.VMEM((GBA, 16), jnp.float32),
            pltpu.VMEM((GBA, 16), jnp.float32),
            pltpu.VMEM((GBA, 16), jnp.float32),
            pltpu.VMEM((GBA, 16), jnp.float32),
            pltpu.VMEM((GBA, 16), jnp.float32),
            pltpu.VMEM((GBA, 16), jnp.float32),
            pltpu.VMEM_SHARED((NP, 16), jnp.float32),
            pltpu.SemaphoreType.DMA,
            pltpu.SemaphoreType.DMA,
        ],
    )
    def sc_att(src_hbm, dst_hbm, asrc_hbm, adst_hbm, w_hbm, den_hbm,
               src_v, dst_v, asg0, adg0, asg1, adg1, coef0, coef1,
               den_sh, sem0, sem1):
        c = lax.axis_index("c")
        s = lax.axis_index("s")
        base_n = s * NPT
        tile = c * NS + s
        row0 = tile * RA2
        slots = ((asg0, adg0, coef0, sem0), (asg1, adg1, coef1, sem1))

        def zero_body(i, _):
            asg0[i] = jnp.zeros((16,), jnp.float32)
            return 0
        lax.fori_loop(0, GBA, zero_body, 0)
        for m in range(NPT // GBA):
            pltpu.sync_copy(asg0, den_sh.at[pl.ds(base_n + m * GBA, GBA)])
        plsc.subcore_barrier()

        pltpu.sync_copy(src_hbm.at[pl.ds(row0, RA2)], src_v)
        pltpu.sync_copy(dst_hbm.at[pl.ds(row0, RA2)], dst_v)

        def issue(j, b):
            asg, adg, _, sem = slots[b]
            pltpu.async_copy(asrc_hbm.at[src_v.at[j]], asg, sem)
            pltpu.async_copy(adst_hbm.at[dst_v.at[j]], adg, sem)

        def process(j, b):
            asg, adg, coef, sem = slots[b]
            pltpu.make_async_copy(asrc_hbm.at[src_v.at[j]], asg, sem).wait()
            pltpu.make_async_copy(adst_hbm.at[dst_v.at[j]], adg, sem).wait()

            def att(i, _):
                e = asg[i] + adg[i]
                e = jnp.where(e >= 0.0, e, 0.2 * e)
                coef[i] = jnp.exp(e)
                return 0
            lax.fori_loop(0, GBA, att, 0)
            pltpu.sync_copy(coef, w_hbm.at[pl.ds((row0 + j) * GBA, GBA)])
            pltpu.sync_copy(coef, den_sh.at[dst_v.at[j]], add=True)

            @pl.when(j + 2 < RA2)
            def _():
                issue(j + 2, b)

        issue(0, 0)
        issue(1, 1)

        def pair(jp, _):
            process(2 * jp, 0)
            process(2 * jp + 1, 1)
            return 0
        lax.fori_loop(0, RA2 // 2, pair, 0)
        plsc.subcore_barrier()

        pltpu.sync_copy(den_sh.at[pl.ds(base_n, NPT)],
                        den_hbm.at[c].at[pl.ds(base_n, NPT)])

    return sc_att


def _make_sc_agg(hc, gb, ch, bf16_feat=False):
    feat_w = HEADS * hc
    fdt = jnp.bfloat16 if bf16_feat else jnp.float32
    rows_t = ET // gb
    n_ch = rows_t // ch

    @functools.partial(
        pl.kernel,
        mesh=_sc_mesh(),
        compiler_params=pltpu.CompilerParams(use_tc_tiling_on_sc=False,
                                             needs_layout_passes=False),
        out_type=jax.ShapeDtypeStruct((NC, NP, 128), jnp.float32),
        scratch_types=[
            pltpu.VMEM((ch, gb), jnp.int32),
            pltpu.VMEM((ch, gb), jnp.int32),
            pltpu.VMEM((gb, 16), jnp.float32),
            pltpu.VMEM((gb, 16), jnp.float32),
            pltpu.VMEM((gb, 16), jnp.float32),
            pltpu.VMEM((gb, 16), jnp.float32),
            pltpu.VMEM((gb, feat_w), fdt),
            pltpu.VMEM((gb, feat_w), fdt),
            pltpu.VMEM((gb, 128), jnp.float32),
            pltpu.VMEM((gb, 128), jnp.float32),
            pltpu.VMEM_SHARED((NP, 128), jnp.float32),
            pltpu.SemaphoreType.DMA,
            pltpu.SemaphoreType.DMA,
            pltpu.SemaphoreType.DMA,
            pltpu.SemaphoreType.DMA,
        ],
    )
    def sc_agg(src_hbm, dst_hbm, w_hbm, rden_hbm, feat_hbm, out_hbm,
               src_v, dst_v, wv0, wv1, rdg0, rdg1, g0, g1, ob0, ob1,
               out_sh, sem0, sem1, ssem0, ssem1):
        c = lax.axis_index("c")
        s = lax.axis_index("s")
        base_n = s * NPT
        tile = c * NS + s
        e0 = tile * ET
        slots = ((wv0, rdg0, g0, ob0, sem0, ssem0),
                 (wv1, rdg1, g1, ob1, sem1, ssem1))

        def zero_body(i, _):
            z = jnp.zeros((16,), jnp.float32)
            for k in range(8):
                ob0[i, pl.ds(16 * k, 16)] = z
            return 0
        lax.fori_loop(0, gb, zero_body, 0)
        for m in range(NPT // gb):
            pltpu.sync_copy(ob0, out_sh.at[pl.ds(base_n + m * gb, gb)])
        plsc.subcore_barrier()

        def issue(jj, cc, b):
            wv, rdg, g, _, sem, _ = slots[b]
            ebase = e0 + (cc * ch + jj) * gb
            pltpu.async_copy(w_hbm.at[pl.ds(ebase, gb)], wv, sem)
            pltpu.async_copy(rden_hbm.at[dst_v.at[jj]], rdg, sem)
            pltpu.async_copy(feat_hbm.at[src_v.at[jj]], g, sem)

        def process(jj, cc, b):
            wv, rdg, g, ob, sem, ssem = slots[b]
            dr = dst_v.at[jj]
            ebase = e0 + (cc * ch + jj) * gb
            pltpu.make_async_copy(w_hbm.at[pl.ds(ebase, gb)], wv, sem).wait()
            pltpu.make_async_copy(rden_hbm.at[dr], rdg, sem).wait()
            pltpu.make_async_copy(feat_hbm.at[src_v.at[jj]], g, sem).wait()

            def norm(i, _):
                wv[i] = wv[i] * rdg[i]
                return 0
            lax.fori_loop(0, gb, norm, 0)

            @pl.when(jj >= 2)
            def _():
                pltpu.make_async_copy(ob, out_sh.at[dr], ssem).wait()

            def edge(i, _):
                cv = wv[i]
                cs = [cv[h] for h in range(HEADS)]
                if hc == 16:
                    for k in range(8):
                        ob[i, pl.ds(16 * k, 16)] = g[i, pl.ds(16 * k, 16)] * cs[k]
                elif not bf16_feat:
                    for k in range(8):
                        acc = g[i, pl.ds(16 * k, 16)] * cs[0]
                        for h in range(1, HEADS):
                            acc = acc + g[i, pl.ds(128 * h + 16 * k, 16)] * cs[h]
                        ob[i, pl.ds(16 * k, 16)] = acc
                else:
                    for m in range(4):
                        lo, hi = plsc.unpack(
                            g[i, pl.ds(32 * m, 32)],
                            format=plsc.PackFormat.INTERLEAVED)
                        acc_a = lo * cs[0]
                        acc_b = hi * cs[0]
                        for h in range(1, HEADS):
                            lo, hi = plsc.unpack(
                                g[i, pl.ds(128 * h + 32 * m, 32)],
                                format=plsc.PackFormat.INTERLEAVED)
                            acc_a = acc_a + lo * cs[h]
                            acc_b = acc_b + hi * cs[h]
                        ob[i, pl.ds(32 * m, 16)] = acc_a
                        ob[i, pl.ds(32 * m + 16, 16)] = acc_b
                return 0
            lax.fori_loop(0, gb, edge, 0)
            pltpu.async_copy(ob, out_sh.at[dr], ssem, add=True)

            @pl.when(jj + 2 < ch)
            def _():
                issue(jj + 2, cc, b)

        def chunk(cc, _):
            pltpu.sync_copy(src_hbm.at[pl.ds(tile * rows_t + cc * ch, ch)], src_v)
            pltpu.sync_copy(dst_hbm.at[pl.ds(tile * rows_t + cc * ch, ch)], dst_v)
            issue(0, cc, 0)
            issue(1, cc, 1)

            def pair(jp, _):
                process(2 * jp, cc, 0)
                process(2 * jp + 1, cc, 1)
                return 0
            lax.fori_loop(0, ch // 2, pair, 0)
            pltpu.make_async_copy(ob0, out_sh.at[dst_v.at[ch - 2]], ssem0).wait()
            pltpu.make_async_copy(ob1, out_sh.at[dst_v.at[ch - 1]], ssem1).wait()
            return 0
        lax.fori_loop(0, n_ch, chunk, 0)
        plsc.subcore_barrier()

        pltpu.sync_copy(out_sh.at[pl.ds(base_n, NPT)],
                        out_hbm.at[c].at[pl.ds(base_n, NPT)])

    return sc_agg


@functools.lru_cache(maxsize=None)
def _sc_kernels(layer):
    if layer == 1:
        return _make_sc_att(), _make_sc_agg(16, 32, 48)
    return _make_sc_att(), _make_sc_agg(128, 32, 48, bf16_feat=True)


_D2 = HEADS * 128
_r = np.arange(_D2)
_PERM2 = (_r // 32) * 32 + np.where(_r % 2 == 0, (_r % 32) // 2,
                                    16 + (_r % 32) // 2)


def _block_diag(att, hc):
    d = HEADS * hc
    rows = jnp.arange(d)
    return jnp.zeros((d, 16), jnp.float32).at[rows, rows // hc].set(att.reshape(d))


@jax.jit
def kernel(x, edge_index, W1, att_src1, att_dst1, b1, W2, att_src2, att_dst2, b2):
    xp = jnp.zeros((NP, 128), jnp.float32).at[:N_NODES].set(x)
    loop = jnp.arange(N_NODES, dtype=jnp.int32)
    pad = TRASH + jnp.arange(EP - E_SL, dtype=jnp.int32) % (NP - N_NODES)
    src = jnp.concatenate([edge_index[0], loop, pad]).reshape(RB, GBA)
    dst = jnp.concatenate([edge_index[1], loop, pad]).reshape(RB, GBA)
    src32 = src.reshape(RB * 2, 32)
    dst32 = dst.reshape(RB * 2, 32)

    att1, agg1 = _sc_kernels(1)
    att2, agg2 = _sc_kernels(2)

    h1, a1s, a1d = _tc1(xp, W1, _block_diag(att_src1, 16), _block_diag(att_dst1, 16))
    w1, d1 = att1(src, dst, a1s, a1d)
    rd1 = _tc_rden(1.0)(d1[0], d1[1])
    p1 = agg1(src32, dst32, w1, rd1, h1)
    h2, a2s, a2d = _tc2(p1[0], p1[1], b1.reshape(1, 128), W2[:, _PERM2],
                        _block_diag(att_src2.reshape(_D2)[_PERM2], 128),
                        _block_diag(att_dst2.reshape(_D2)[_PERM2], 128))
    w2, d2 = att2(src, dst, a2s, a2d)
    rd2 = _tc_rden(1.0 / HEADS)(d2[0], d2[1])
    p2 = agg2(src32, dst32, w2, rd2, h2)
    return p2[0, :N_NODES] + p2[1, :N_NODES] + b2
